# baseline (speedup 1.0000x reference)
import dataclasses

import jax
import jax.numpy as jnp
from jax import lax
from jax.experimental import pallas as pl
from jax.experimental.pallas import tpu as pltpu
from jax.experimental.pallas import tpu_sc as plsc

_ROWS = 9984
_W = 2048
_NW = 32
_RPT = _ROWS // _NW
_U = 8
_NU = _RPT // _U


def _compute(a, b, o):
    for r in range(_U):
        @pl.loop(0, _W, step=16, unroll=4)
        def _(c, r=r):
            o[r, pl.ds(c, 16)] = a[r, pl.ds(c, 16)] - b[r, pl.ds(c, 16)]


def _sc_body(par_hbm, x_hbm, o_hbm, par_v,
             a0, b0, o0, a1, b1, o1,
             sa0, sb0, so0, sa1, sb1, so1):
    c = lax.axis_index("c")
    s = lax.axis_index("s")
    wid = s * 2 + c
    r0 = wid * _RPT
    pltpu.sync_copy(par_hbm.at[pl.ds(r0, _RPT)], par_v)

    def in_copies(u, ab, bb, sa, sb):
        rr = r0 + u * _U
        ca = pltpu.make_async_copy(x_hbm.at[pl.ds(rr, _U)], ab, sa)
        cb = pltpu.make_async_copy(x_hbm.at[par_v.at[pl.ds(u * _U, _U)]],
                                   bb, sb)
        return ca, cb

    def out_copy(u, ob, so):
        rr = r0 + u * _U
        return pltpu.make_async_copy(ob, o_hbm.at[pl.ds(rr, _U)], so)

    def stage(u, ab, bb, ob, sa, sb, so):
        ca, cb = in_copies(u, ab, bb, sa, sb)
        ca.wait()
        cb.wait()
        @pl.when(u >= 2)
        def _():
            out_copy(u - 2, ob, so).wait()
        _compute(ab, bb, ob)
        out_copy(u, ob, so).start()
        @pl.when(u + 2 < _NU)
        def _():
            na, nb = in_copies(u + 2, ab, bb, sa, sb)
            na.start()
            nb.start()

    ca, cb = in_copies(0, a0, b0, sa0, sb0)
    ca.start()
    cb.start()
    ca, cb = in_copies(1, a1, b1, sa1, sb1)
    ca.start()
    cb.start()

    @pl.loop(0, _NU - 1, step=2)
    def _(u):
        stage(u, a0, b0, o0, sa0, sb0, so0)
        stage(u + 1, a1, b1, o1, sa1, sb1, so1)

    stage(_NU - 1, a0, b0, o0, sa0, sb0, so0)
    out_copy(_NU - 2, o1, so1).wait()
    out_copy(_NU - 1, o0, so0).wait()


@jax.jit
def kernel(positions, parents):
    B, T, J, C = positions.shape
    D = J * C
    x = positions.transpose(2, 3, 0, 1).reshape(D * B, T)

    perm = (parents.astype(jnp.int32)[:, None] * C
            + jnp.arange(C, dtype=jnp.int32)[None, :]).reshape(D)
    par_row = (perm[:, None] * B
               + jnp.arange(B, dtype=jnp.int32)[None, :]).reshape(D * B)

    cp = pltpu.CompilerParams()
    if "needs_layout_passes" in pltpu.CompilerParams.__dataclass_fields__:
        cp = dataclasses.replace(cp, needs_layout_passes=False)
    sc_call = pl.kernel(
        _sc_body,
        out_type=jax.ShapeDtypeStruct((D * B, T), jnp.float32),
        mesh=plsc.VectorSubcoreMesh(core_axis_name="c", subcore_axis_name="s"),
        scratch_types=[
            pltpu.VMEM((_RPT,), jnp.int32),
            pltpu.VMEM((_U, _W), jnp.float32),
            pltpu.VMEM((_U, _W), jnp.float32),
            pltpu.VMEM((_U, _W), jnp.float32),
            pltpu.VMEM((_U, _W), jnp.float32),
            pltpu.VMEM((_U, _W), jnp.float32),
            pltpu.VMEM((_U, _W), jnp.float32),
            pltpu.SemaphoreType.DMA,
            pltpu.SemaphoreType.DMA,
            pltpu.SemaphoreType.DMA,
            pltpu.SemaphoreType.DMA,
            pltpu.SemaphoreType.DMA,
            pltpu.SemaphoreType.DMA,
        ],
        compiler_params=cp,
    )
    out = sc_call(par_row, x)
    return out.reshape(J, C, B, T).transpose(2, 3, 0, 1)


# speedup vs baseline: 2.8826x; 2.8826x over previous
import jax
import jax.numpy as jnp
from jax import lax
from jax.experimental import pallas as pl
from jax.experimental.pallas import tpu as pltpu


def _offsets_body(perm_ref, x_ref, o_ref):
    def step(i, carry):
        p = perm_ref[i]
        o_ref[i] = x_ref[i] - x_ref[p]
        return carry

    lax.fori_loop(0, x_ref.shape[0], step, 0, unroll=4)


@jax.jit
def kernel(positions, parents):
    B, T, J, C = positions.shape
    D = J * C
    x = positions.transpose(2, 3, 0, 1).reshape(D, B, T)

    perm = (parents.astype(jnp.int32)[:, None] * C
            + jnp.arange(C, dtype=jnp.int32)[None, :]).reshape(D)

    RB, CT = 8, 2048
    out = pl.pallas_call(
        _offsets_body,
        grid_spec=pltpu.PrefetchScalarGridSpec(
            num_scalar_prefetch=1,
            grid=(B // RB, T // CT),
            in_specs=[pl.BlockSpec((D, RB, CT), lambda i, j, perm_ref: (0, i, j))],
            out_specs=pl.BlockSpec((D, RB, CT), lambda i, j, perm_ref: (0, i, j)),
        ),
        out_shape=jax.ShapeDtypeStruct((D, B, T), jnp.float32),
    )(perm, x)
    return out.reshape(J, C, B, T).transpose(2, 3, 0, 1)


# speedup vs baseline: 2.8985x; 1.0055x over previous
import jax
import jax.numpy as jnp
from jax import lax
from jax.experimental import pallas as pl
from jax.experimental.pallas import tpu as pltpu


def _offsets_body(perm_ref, x_ref, o_ref):
    def step(i, carry):
        p = perm_ref[i]
        o_ref[i] = x_ref[i] - x_ref[p]
        return carry

    lax.fori_loop(0, x_ref.shape[0], step, 0, unroll=8)


@jax.jit
def kernel(positions, parents):
    B, T, J, C = positions.shape
    D = J * C
    x = positions.transpose(2, 3, 0, 1).reshape(D, B, T)

    perm = (parents.astype(jnp.int32)[:, None] * C
            + jnp.arange(C, dtype=jnp.int32)[None, :]).reshape(D)

    RB, CT = 8, 2048
    out = pl.pallas_call(
        _offsets_body,
        grid_spec=pltpu.PrefetchScalarGridSpec(
            num_scalar_prefetch=1,
            grid=(B // RB, T // CT),
            in_specs=[pl.BlockSpec((D, RB, CT), lambda i, j, perm_ref: (0, i, j))],
            out_specs=pl.BlockSpec((D, RB, CT), lambda i, j, perm_ref: (0, i, j)),
        ),
        out_shape=jax.ShapeDtypeStruct((D, B, T), jnp.float32),
    )(perm, x)
    return out.reshape(J, C, B, T).transpose(2, 3, 0, 1)
